# baseline (speedup 1.0000x reference)
import functools

import jax
import jax.numpy as jnp
from jax import lax
from jax.experimental import pallas as pl
from jax.experimental.pallas import tpu as pltpu
from jax.experimental.pallas import tpu_sc as plsc

_N = 10000
_E = 160000
_ET = _E + _N
_NW = 32
_EB = 128
_NG = 16
_GS = 12032
_NS = _NG * _GS
_NBS = 47
_NBG = 94
_FB = 128
_NBF = 40
_FPW = _NBF * _FB
_FP = _NW * _FPW
_BN = 1000
_NP = 10240


def _tc1_body(x_ref, w_ref, as_ref, ad_ref, h_ref, es_ref, ed_ref):
    j = pl.program_id(1)
    head = j // 2
    blk = jnp.dot(x_ref[...], w_ref[0], preferred_element_type=jnp.float32)
    h_ref[0] = blk
    pes = jnp.sum(blk * as_ref[0], axis=1, keepdims=True)
    ped = jnp.sum(blk * ad_ref[0], axis=1, keepdims=True)
    onehot = (lax.broadcasted_iota(jnp.int32, (1, 3), 1) == head
              ).astype(jnp.float32)

    @pl.when(j == 0)
    def _():
        es_ref[...] = jnp.zeros_like(es_ref)
        ed_ref[...] = jnp.zeros_like(ed_ref)

    es_ref[...] += pes * onehot
    ed_ref[...] += ped * onehot


def _tc1(x, w, a_s, a_d):
    grid = (_N // _BN, 6)
    return pl.pallas_call(
        _tc1_body,
        grid=grid,
        in_specs=[
            pl.BlockSpec((_BN, 256), lambda i, j: (i, 0)),
            pl.BlockSpec((1, 256, 128), lambda i, j: (j, 0, 0)),
            pl.BlockSpec((1, 1, 128), lambda i, j: (j, 0, 0)),
            pl.BlockSpec((1, 1, 128), lambda i, j: (j, 0, 0)),
        ],
        out_specs=[
            pl.BlockSpec((1, _BN, 128), lambda i, j: (j, i, 0)),
            pl.BlockSpec((_BN, 3), lambda i, j: (i, 0)),
            pl.BlockSpec((_BN, 3), lambda i, j: (i, 0)),
        ],
        out_shape=[
            jax.ShapeDtypeStruct((6, _N, 128), jnp.float32),
            jax.ShapeDtypeStruct((_N, 3), jnp.float32),
            jax.ShapeDtypeStruct((_N, 3), jnp.float32),
        ],
    )(x, w.reshape(256, 6, 128).transpose(1, 0, 2),
      a_s.reshape(6, 1, 128), a_d.reshape(6, 1, 128))


def _tc2_body(x_ref, b1_ref, w_ref, as_ref, ad_ref, h_ref, es_ref, ed_ref):
    j = pl.program_id(1)
    acc = jnp.zeros((_BN, 128), jnp.float32)
    for c in range(6):
        xc = jnp.maximum(
            x_ref[c] + b1_ref[0, c * 128:(c + 1) * 128][None, :], 0.0)
        acc += jnp.dot(xc, w_ref[0, c * 128:(c + 1) * 128, :],
                       preferred_element_type=jnp.float32)
    h_ref[0] = acc
    pes = jnp.sum(acc * as_ref[0], axis=1, keepdims=True)
    ped = jnp.sum(acc * ad_ref[0], axis=1, keepdims=True)

    @pl.when(j == 0)
    def _():
        es_ref[...] = jnp.zeros_like(es_ref)
        ed_ref[...] = jnp.zeros_like(ed_ref)

    es_ref[...] += pes
    ed_ref[...] += ped


def _tc2(x3, b1, w, a_s, a_d):
    grid = (_N // _BN, 2)
    return pl.pallas_call(
        _tc2_body,
        grid=grid,
        in_specs=[
            pl.BlockSpec((6, _BN, 128), lambda i, j: (0, i, 0)),
            pl.BlockSpec((1, 768), lambda i, j: (0, 0)),
            pl.BlockSpec((1, 768, 128), lambda i, j: (j, 0, 0)),
            pl.BlockSpec((1, 1, 128), lambda i, j: (j, 0, 0)),
            pl.BlockSpec((1, 1, 128), lambda i, j: (j, 0, 0)),
        ],
        out_specs=[
            pl.BlockSpec((1, _BN, 128), lambda i, j: (j, i, 0)),
            pl.BlockSpec((_BN, 1), lambda i, j: (i, 0)),
            pl.BlockSpec((_BN, 1), lambda i, j: (i, 0)),
        ],
        out_shape=[
            jax.ShapeDtypeStruct((2, _N, 128), jnp.float32),
            jax.ShapeDtypeStruct((_N, 1), jnp.float32),
            jax.ShapeDtypeStruct((_N, 1), jnp.float32),
        ],
    )(x3, b1.reshape(1, 768), w.reshape(768, 2, 128).transpose(1, 0, 2),
      a_s.reshape(2, 1, 128), a_d.reshape(2, 1, 128))


def _tc4_body(x_ref, o_ref):
    o_ref[...] = x_ref[...]


def _tc4(sig):
    grid = (_FP // 20480,)
    return pl.pallas_call(
        _tc4_body,
        grid=grid,
        in_specs=[pl.BlockSpec((20480,), lambda i: (i,))],
        out_specs=pl.BlockSpec((20480,), lambda i: (i,)),
        out_shape=jax.ShapeDtypeStruct((_FP,), jnp.float32),
    )(sig)


def _tc3_body(x_ref, b2_ref, o_ref):
    for c in range(2):
        o_ref[:, c * 128:(c + 1) * 128] = (
            x_ref[c] + b2_ref[0, c * 128:(c + 1) * 128][None, :])


def _tc3(x3, b2):
    grid = (_N // _BN,)
    return pl.pallas_call(
        _tc3_body,
        grid=grid,
        in_specs=[
            pl.BlockSpec((2, _BN, 128), lambda i: (0, i, 0)),
            pl.BlockSpec((1, 256), lambda i: (0, 0)),
        ],
        out_specs=pl.BlockSpec((_BN, 256), lambda i: (i, 0)),
        out_shape=jax.ShapeDtypeStruct((_N, 256), jnp.float32),
    )(x3, b2.reshape(1, 256))


def _dpr(heads):
    return 256 if heads == 3 else 128


@functools.cache
def _make_sca(heads):
    dpr = _dpr(heads)
    stripe = dpr // 16
    nh = _N * heads
    nhp = ((nh + 127) // 128) * 128
    mesh = plsc.VectorSubcoreMesh(core_axis_name="c", subcore_axis_name="s")

    @functools.partial(
        pl.kernel, mesh=mesh,
        compiler_params=pltpu.CompilerParams(needs_layout_passes=False),
        out_type=(
            jax.ShapeDtypeStruct((heads, _NW, _NBS, _EB), jnp.float32),
            jax.ShapeDtypeStruct((2, dpr, 128), jnp.float32),
        ),
        scratch_types=[
            pltpu.VMEM((nhp,), jnp.float32),
            pltpu.VMEM((nhp,), jnp.float32),
            pltpu.VMEM((dpr, 128), jnp.float32),
            pltpu.VMEM((_NBS, _EB), jnp.int32),
            pltpu.VMEM((_NBS, _EB), jnp.int32),
            pltpu.VMEM((_NBS, _EB), jnp.float32),
            pltpu.VMEM((max(heads, 2), _EB), jnp.float32),
            pltpu.VMEM((dpr // 128, _EB), jnp.int32),
            pltpu.VMEM_SHARED((dpr, 128), jnp.float32),
        ],
    )
    def sca(es_h, ed_h, src_h, dst_h, vm_h, zeros_h, rows_idx_h, ex_h, den_h,
            es_v, ed_v, den_v, src_w, dst_w, vm_w, ex_b, ridx_w, den_sh):
        cid = lax.axis_index("c")
        sid = lax.axis_index("s")
        wid = sid * 2 + cid
        pltpu.sync_copy(es_h, es_v.at[pl.ds(0, nh)])
        pltpu.sync_copy(ed_h, ed_v.at[pl.ds(0, nh)])
        pltpu.sync_copy(zeros_h.at[pl.ds(0, dpr), :], den_v)
        pltpu.sync_copy(src_h.at[wid], src_w)
        pltpu.sync_copy(dst_h.at[wid], dst_w)
        pltpu.sync_copy(vm_h.at[wid], vm_w)
        pltpu.sync_copy(rows_idx_h.at[pl.ds(0, dpr // 128)], ridx_w)

        def grp(g, b):
            off = g * 16
            srcv = src_w[b, pl.ds(off, 16)]
            dstv = dst_w[b, pl.ds(off, 16)]
            vmv = vm_w[b, pl.ds(off, 16)]
            for h in range(heads):
                esv = plsc.load_gather(es_v, [srcv * heads + h])
                edv = plsc.load_gather(ed_v, [dstv * heads + h])
                e = esv + edv
                e = jnp.maximum(e, 0.2 * e)
                ex = jnp.exp(e) * vmv
                ex_b[h, pl.ds(off, 16)] = ex
                idx = dstv * heads + h
                plsc.addupdate_scatter(den_v, [idx >> 7, idx & 127], ex)
            return b

        def batch(b, _):
            lax.fori_loop(0, _EB // 16, grp, b)
            for h in range(heads):
                pltpu.sync_copy(ex_b.at[h], ex_h.at[h, wid, b])
            return 0

        lax.fori_loop(0, _NBS, batch, 0)
        pltpu.sync_copy(zeros_h.at[pl.ds(0, stripe), :],
                        den_sh.at[pl.ds(sid * stripe, stripe), :])
        plsc.subcore_barrier()
        for blk in range(dpr // 128):
            pltpu.sync_copy(den_v.at[pl.ds(blk * _EB, _EB), :],
                            den_sh.at[ridx_w.at[blk]], add=True)
        plsc.subcore_barrier()
        pltpu.sync_copy(den_sh.at[pl.ds(sid * stripe, stripe), :],
                        den_h.at[cid, pl.ds(sid * stripe, stripe), :])

    return sca


@functools.cache
def _make_scal(heads):
    dpr = _dpr(heads)
    mesh = plsc.VectorSubcoreMesh(core_axis_name="c", subcore_axis_name="s")

    @functools.partial(
        pl.kernel, mesh=mesh,
        compiler_params=pltpu.CompilerParams(needs_layout_passes=False),
        out_type=jax.ShapeDtypeStruct((heads, _NW, _NBS, _EB), jnp.float32),
        scratch_types=[
            pltpu.VMEM((dpr, 128), jnp.float32),
            pltpu.VMEM((16, 128), jnp.float32),
            pltpu.VMEM((_NBS, _EB), jnp.int32),
            pltpu.VMEM((_NBS, _EB), jnp.float32),
            pltpu.VMEM((_NBS, _EB), jnp.float32),
        ],
    )
    def scal(ex_h, den_h, dst_h, al_h,
             den_a, den_blk, dst_w, ex_w, al_w):
        cid = lax.axis_index("c")
        sid = lax.axis_index("s")
        wid = sid * 2 + cid
        pltpu.sync_copy(den_h.at[0], den_a)
        pltpu.sync_copy(dst_h.at[wid], dst_w)

        def dsum(blk, _):
            pltpu.sync_copy(den_h.at[1, pl.ds(blk * 16, 16)], den_blk)
            for r in range(16):
                for q in range(8):
                    sl = pl.ds(q * 16, 16)
                    den_a[blk * 16 + r, sl] = (den_a[blk * 16 + r, sl]
                                               + den_blk[r, sl])
            return 0

        lax.fori_loop(0, dpr // 16, dsum, 0)
        for h in range(heads):
            pltpu.sync_copy(ex_h.at[h, wid], ex_w)

            def agrp(g, b):
                off = g * 16
                dstv = dst_w[b, pl.ds(off, 16)]
                exv = ex_w[b, pl.ds(off, 16)]
                idx = dstv * heads + h
                denv = plsc.load_gather(den_a, [idx >> 7, idx & 127])
                al_w[b, pl.ds(off, 16)] = exv / (denv + 1e-16)
                return b

            def abatch(b, _):
                lax.fori_loop(0, _EB // 16, agrp, b)
                return 0

            lax.fori_loop(0, _NBS, abatch, 0)
            pltpu.sync_copy(al_w, al_h.at[h, wid])

    return scal


@functools.cache
def _make_scb(nchunks, heads):
    cs = nchunks // 2
    hdiv = nchunks // heads
    rows_pt = _NP // 16
    mesh = plsc.VectorSubcoreMesh(core_axis_name="c", subcore_axis_name="s")

    @functools.partial(
        pl.kernel, mesh=mesh,
        compiler_params=pltpu.CompilerParams(needs_layout_passes=False),
        out_type=jax.ShapeDtypeStruct((nchunks, _NP, 128), jnp.float32),
        scratch_types=[
            pltpu.VMEM((_NBS, _EB), jnp.int32),
            pltpu.VMEM((_NBS, _EB), jnp.int32),
            pltpu.VMEM((2, _EB), jnp.float32),
            pltpu.VMEM((_EB, 128), jnp.float32),
            pltpu.VMEM((_EB, 128), jnp.float32),
            pltpu.VMEM_SHARED((_NP, 128), jnp.float32),
            pltpu.SemaphoreType.DMA,
            pltpu.SemaphoreType.DMA,
            pltpu.SemaphoreType.DMA,
            pltpu.SemaphoreType.DMA,
        ],
    )
    def scb(h3_h, src_h, dst_h, al_h, z2_h, o3_h,
            src_w, dst_w, al_b, rows_a, rows_b, acc_sh,
            sem_a, sem_b, sem_sa, sem_sb):
        cid = lax.axis_index("c")
        sid = lax.axis_index("s")

        def chunk(k, _):
            c = cid * cs + k
            h = c // hdiv
            pltpu.sync_copy(z2_h, acc_sh.at[pl.ds(sid * rows_pt, rows_pt)])
            plsc.subcore_barrier()

            def half(b1, _):
                pltpu.sync_copy(src_h.at[2 * sid + b1], src_w)
                pltpu.sync_copy(dst_h.at[2 * sid + b1], dst_w)
                hc = h3_h.at[c]
                pltpu.async_copy(hc.at[src_w.at[0]], rows_a, sem_a)

                def scale_rows(b, rows_v):
                    pltpu.sync_copy(al_h.at[h, 2 * sid + b1, pl.ds(b, 1)],
                                    al_b.at[pl.ds(0, 1)])

                    def scale(g, _):
                        av16 = al_b[0, pl.ds(g * 16, 16)]
                        for j in range(16):
                            avv = jnp.full((16,), av16[j], jnp.float32)
                            e = g * 16 + j
                            for q in range(8):
                                rows_v[e, pl.ds(q * 16, 16)] = (
                                    rows_v[e, pl.ds(q * 16, 16)] * avv)
                        return 0

                    lax.fori_loop(0, _EB // 16, scale, 0)

                def pair(i, _):
                    @pl.when(i > 0)
                    def _():
                        pltpu.make_async_copy(
                            rows_b, acc_sh.at[dst_w.at[0]], sem_sb).wait()
                    pltpu.async_copy(hc.at[src_w.at[2 * i + 1]], rows_b,
                                     sem_b)
                    pltpu.make_async_copy(hc.at[src_w.at[2 * i]], rows_a,
                                          sem_a).wait()
                    scale_rows(2 * i, rows_a)
                    pltpu.async_copy(rows_a, acc_sh.at[dst_w.at[2 * i]],
                                     sem_sa, add=True)
                    pltpu.make_async_copy(hc.at[src_w.at[2 * i + 1]],
                                          rows_b, sem_b).wait()
                    scale_rows(2 * i + 1, rows_b)
                    pltpu.async_copy(rows_b,
                                     acc_sh.at[dst_w.at[2 * i + 1]],
                                     sem_sb, add=True)
                    pltpu.make_async_copy(rows_a, acc_sh.at[dst_w.at[0]],
                                          sem_sa).wait()
                    pltpu.async_copy(hc.at[src_w.at[2 * i + 2]], rows_a,
                                     sem_a)
                    return 0

                lax.fori_loop(0, (_NBS - 1) // 2, pair, 0)
                pltpu.make_async_copy(rows_b, acc_sh.at[dst_w.at[0]],
                                      sem_sb).wait()
                pltpu.make_async_copy(hc.at[src_w.at[_NBS - 1]], rows_a,
                                      sem_a).wait()
                scale_rows(_NBS - 1, rows_a)
                pltpu.sync_copy(rows_a, acc_sh.at[dst_w.at[_NBS - 1]],
                                add=True)
                return 0

            lax.fori_loop(0, 2, half, 0)
            plsc.subcore_barrier()
            pltpu.sync_copy(acc_sh.at[pl.ds(sid * rows_pt, rows_pt)],
                            o3_h.at[c, pl.ds(sid * rows_pt, rows_pt)])
            plsc.subcore_barrier()
            return 0

        lax.fori_loop(0, cs, chunk, 0)

    return scb


@functools.cache
def _make_scc():
    mesh = plsc.VectorSubcoreMesh(core_axis_name="c", subcore_axis_name="s")

    @functools.partial(
        pl.kernel, mesh=mesh,
        compiler_params=pltpu.CompilerParams(needs_layout_passes=False),
        out_type=jax.ShapeDtypeStruct((_FP,), jnp.float32),
        scratch_types=[
            pltpu.VMEM((_NBF, _FB), jnp.int32),
            pltpu.VMEM((_NBF, _FB), jnp.int32),
            pltpu.VMEM((_FB, 256), jnp.float32),
            pltpu.VMEM((_FB, 256), jnp.float32),
            pltpu.VMEM((_FB * 16,), jnp.float32),
            pltpu.VMEM((_FPW,), jnp.float32),
            pltpu.SemaphoreType.DMA,
            pltpu.SemaphoreType.DMA,
        ],
    )
    def scc(h2_h, e0_h, e1_h, sig_h,
            e0_w, e1_w, buf_a, buf_b, pb, out_w, sem_a, sem_b):
        cid = lax.axis_index("c")
        sid = lax.axis_index("s")
        wid = sid * 2 + cid
        pltpu.sync_copy(e0_h.at[wid], e0_w)
        pltpu.sync_copy(e1_h.at[wid], e1_w)

        def batch(b, _):
            da = pltpu.async_copy(h2_h.at[e0_w.at[b]], buf_a, sem_a)
            db = pltpu.async_copy(h2_h.at[e1_w.at[b]], buf_b, sem_b)
            da.wait()
            db.wait()

            def dot(e, _):
                acc = jnp.zeros((16,), jnp.float32)
                for q in range(16):
                    acc = acc + (buf_a[e, pl.ds(q * 16, 16)]
                                 * buf_b[e, pl.ds(q * 16, 16)])
                pb[pl.ds(e * 16, 16)] = acc
                return 0

            lax.fori_loop(0, _FB, dot, 0, unroll=4)

            def red(g, _):
                row0 = (g * 16 + lax.iota(jnp.int32, 16)) * 16
                s = jnp.zeros((16,), jnp.float32)
                for q in range(16):
                    s = s + plsc.load_gather(pb, [row0 + q])
                sig = 1.0 / (1.0 + jnp.exp(-s))
                out_w[pl.ds(b * _FB + g * 16, 16)] = sig
                return 0

            lax.fori_loop(0, _FB // 16, red, 0)
            return 0

        lax.fori_loop(0, _NBF, batch, 0)
        pltpu.sync_copy(out_w, sig_h.at[pl.ds(wid * _FPW, _FPW)])

    return scc


def _pad_idx(x, total):
    out = jnp.zeros((total,), jnp.int32)
    return out.at[:x.shape[0]].set(x.astype(jnp.int32))


def _group_edges(src, dst):
    ge = (dst // 640).astype(jnp.int32)
    cnt = jnp.bincount(ge, length=_NG)
    bnd = jnp.cumsum(_GS - cnt)
    gpad = jnp.searchsorted(bnd, jnp.arange(_NS - _ET), side="right"
                            ).astype(jnp.int32)
    g_all = jnp.concatenate([ge, gpad])
    ispad = jnp.concatenate([jnp.zeros((_ET,), jnp.int32),
                             jnp.ones((_NS - _ET,), jnp.int32)])
    perm = jnp.argsort(g_all * 2 + ispad, stable=True)
    src_all = jnp.concatenate([src.astype(jnp.int32),
                               jnp.zeros((_NS - _ET,), jnp.int32)])
    dst_all = jnp.concatenate([dst.astype(jnp.int32), gpad * 640])
    vm_all = jnp.concatenate([jnp.ones((_ET,), jnp.float32),
                              jnp.zeros((_NS - _ET,), jnp.float32)])
    shape = (_NW, _NBS, _EB)
    return (src_all[perm].reshape(shape), dst_all[perm].reshape(shape),
            vm_all[perm].reshape(shape))


def kernel(Features, A, E, W1, a1_src, a1_dst, b1, W2, a2_src, a2_dst, b2):
    loops = jnp.arange(_N, dtype=A.dtype)
    src = jnp.concatenate([A[0], loops])
    dst = jnp.concatenate([A[1], loops])
    src3, dst3, vm3 = _group_edges(src, dst)
    e03 = _pad_idx(E[0], _FP).reshape(_NW, _NBF, _FB)
    e13 = _pad_idx(E[1], _FP).reshape(_NW, _NBF, _FB)
    zeros_dn = jnp.zeros((256, 128), jnp.float32)
    zeros2 = jnp.zeros((_NP // 16, 128), jnp.float32)
    rows_idx = jnp.arange(256, dtype=jnp.int32).reshape(2, 128)

    h3_1, es1, ed1 = _tc1(Features, W1, a1_src, a1_dst)
    ex1, den1 = _make_sca(3)(es1.reshape(-1), ed1.reshape(-1),
                             src3, dst3, vm3, zeros_dn, rows_idx)
    al1 = _make_scal(3)(ex1, den1, dst3)
    out1 = _make_scb(6, 3)(h3_1, src3, dst3, al1, zeros2)
    h23, es2, ed2 = _tc2(out1, b1, W2, a2_src, a2_dst)
    ex2, den2 = _make_sca(1)(es2.reshape(-1), ed2.reshape(-1),
                             src3, dst3, vm3, zeros_dn, rows_idx)
    al2 = _make_scal(1)(ex2, den2, dst3)
    out2 = _make_scb(2, 1)(h23, src3, dst3, al2, zeros2)
    h2 = _tc3(out2, b2)
    sig = _make_scc()(h2, e03, e13)
    return _tc4(sig)[:_E]


# speedup vs baseline: 1.0147x; 1.0147x over previous
import functools

import jax
import jax.numpy as jnp
from jax import lax
from jax.experimental import pallas as pl
from jax.experimental.pallas import tpu as pltpu
from jax.experimental.pallas import tpu_sc as plsc

_N = 10000
_E = 160000
_ET = _E + _N
_NW = 32
_EB = 128
_NG = 16
_GS = 12032
_NS = _NG * _GS
_NBS = 47
_NBG = 94
_FB = 128
_NBF = 40
_FPW = _NBF * _FB
_FP = _NW * _FPW
_BN = 1000
_NP = 10240


def _tc1_body(x_ref, w_ref, as_ref, ad_ref, h_ref, es_ref, ed_ref):
    j = pl.program_id(1)
    head = j // 2
    blk = jnp.dot(x_ref[...], w_ref[0], preferred_element_type=jnp.float32)
    h_ref[0] = blk
    pes = jnp.sum(blk * as_ref[0], axis=1, keepdims=True)
    ped = jnp.sum(blk * ad_ref[0], axis=1, keepdims=True)
    onehot = (lax.broadcasted_iota(jnp.int32, (1, 3), 1) == head
              ).astype(jnp.float32)

    @pl.when(j == 0)
    def _():
        es_ref[...] = jnp.zeros_like(es_ref)
        ed_ref[...] = jnp.zeros_like(ed_ref)

    es_ref[...] += pes * onehot
    ed_ref[...] += ped * onehot


def _tc1(x, w, a_s, a_d):
    grid = (_N // _BN, 6)
    return pl.pallas_call(
        _tc1_body,
        grid=grid,
        in_specs=[
            pl.BlockSpec((_BN, 256), lambda i, j: (i, 0)),
            pl.BlockSpec((1, 256, 128), lambda i, j: (j, 0, 0)),
            pl.BlockSpec((1, 1, 128), lambda i, j: (j, 0, 0)),
            pl.BlockSpec((1, 1, 128), lambda i, j: (j, 0, 0)),
        ],
        out_specs=[
            pl.BlockSpec((1, _BN, 128), lambda i, j: (j, i, 0)),
            pl.BlockSpec((_BN, 3), lambda i, j: (i, 0)),
            pl.BlockSpec((_BN, 3), lambda i, j: (i, 0)),
        ],
        out_shape=[
            jax.ShapeDtypeStruct((6, _N, 128), jnp.float32),
            jax.ShapeDtypeStruct((_N, 3), jnp.float32),
            jax.ShapeDtypeStruct((_N, 3), jnp.float32),
        ],
    )(x, w.reshape(256, 6, 128).transpose(1, 0, 2),
      a_s.reshape(6, 1, 128), a_d.reshape(6, 1, 128))


def _tc2_body(x_ref, b1_ref, w_ref, as_ref, ad_ref, h_ref, es_ref, ed_ref):
    j = pl.program_id(1)
    acc = jnp.zeros((_BN, 128), jnp.float32)
    for c in range(6):
        xc = jnp.maximum(
            x_ref[c] + b1_ref[0, c * 128:(c + 1) * 128][None, :], 0.0)
        acc += jnp.dot(xc, w_ref[0, c * 128:(c + 1) * 128, :],
                       preferred_element_type=jnp.float32)
    h_ref[0] = acc
    pes = jnp.sum(acc * as_ref[0], axis=1, keepdims=True)
    ped = jnp.sum(acc * ad_ref[0], axis=1, keepdims=True)

    @pl.when(j == 0)
    def _():
        es_ref[...] = jnp.zeros_like(es_ref)
        ed_ref[...] = jnp.zeros_like(ed_ref)

    es_ref[...] += pes
    ed_ref[...] += ped


def _tc2(x3, b1, w, a_s, a_d):
    grid = (_N // _BN, 2)
    return pl.pallas_call(
        _tc2_body,
        grid=grid,
        in_specs=[
            pl.BlockSpec((6, _BN, 128), lambda i, j: (0, i, 0)),
            pl.BlockSpec((1, 768), lambda i, j: (0, 0)),
            pl.BlockSpec((1, 768, 128), lambda i, j: (j, 0, 0)),
            pl.BlockSpec((1, 1, 128), lambda i, j: (j, 0, 0)),
            pl.BlockSpec((1, 1, 128), lambda i, j: (j, 0, 0)),
        ],
        out_specs=[
            pl.BlockSpec((1, _BN, 128), lambda i, j: (j, i, 0)),
            pl.BlockSpec((_BN, 1), lambda i, j: (i, 0)),
            pl.BlockSpec((_BN, 1), lambda i, j: (i, 0)),
        ],
        out_shape=[
            jax.ShapeDtypeStruct((2, _N, 128), jnp.float32),
            jax.ShapeDtypeStruct((_N, 1), jnp.float32),
            jax.ShapeDtypeStruct((_N, 1), jnp.float32),
        ],
    )(x3, b1.reshape(1, 768), w.reshape(768, 2, 128).transpose(1, 0, 2),
      a_s.reshape(2, 1, 128), a_d.reshape(2, 1, 128))


def _tc4_body(x_ref, o_ref):
    o_ref[...] = x_ref[...]


def _tc4(sig):
    grid = (_FP // 20480,)
    return pl.pallas_call(
        _tc4_body,
        grid=grid,
        in_specs=[pl.BlockSpec((20480,), lambda i: (i,))],
        out_specs=pl.BlockSpec((20480,), lambda i: (i,)),
        out_shape=jax.ShapeDtypeStruct((_FP,), jnp.float32),
    )(sig)


def _tc3_body(x_ref, b2_ref, o_ref):
    for c in range(2):
        o_ref[:, c * 128:(c + 1) * 128] = (
            x_ref[c] + b2_ref[0, c * 128:(c + 1) * 128][None, :])


def _tc3(x3, b2):
    grid = (_N // _BN,)
    return pl.pallas_call(
        _tc3_body,
        grid=grid,
        in_specs=[
            pl.BlockSpec((2, _BN, 128), lambda i: (0, i, 0)),
            pl.BlockSpec((1, 256), lambda i: (0, 0)),
        ],
        out_specs=pl.BlockSpec((_BN, 256), lambda i: (i, 0)),
        out_shape=jax.ShapeDtypeStruct((_N, 256), jnp.float32),
    )(x3, b2.reshape(1, 256))


def _dpr(heads):
    return 256 if heads == 3 else 128


@functools.cache
def _make_sca(heads):
    dpr = _dpr(heads)
    stripe = dpr // 16
    nh = _N * heads
    nhp = ((nh + 127) // 128) * 128
    mesh = plsc.VectorSubcoreMesh(core_axis_name="c", subcore_axis_name="s")

    @functools.partial(
        pl.kernel, mesh=mesh,
        compiler_params=pltpu.CompilerParams(needs_layout_passes=False),
        out_type=(
            jax.ShapeDtypeStruct((heads, _NW, _NBS, _EB), jnp.float32),
            jax.ShapeDtypeStruct((2, dpr, 128), jnp.float32),
        ),
        scratch_types=[
            pltpu.VMEM((nhp,), jnp.float32),
            pltpu.VMEM((nhp,), jnp.float32),
            pltpu.VMEM((dpr, 128), jnp.float32),
            pltpu.VMEM((_NBS, _EB), jnp.int32),
            pltpu.VMEM((_NBS, _EB), jnp.int32),
            pltpu.VMEM((_NBS, _EB), jnp.float32),
            pltpu.VMEM((max(heads, 2), _EB), jnp.float32),
            pltpu.VMEM((dpr // 128, _EB), jnp.int32),
            pltpu.VMEM_SHARED((dpr, 128), jnp.float32),
        ],
    )
    def sca(es_h, ed_h, src_h, dst_h, vm_h, zeros_h, rows_idx_h, ex_h, den_h,
            es_v, ed_v, den_v, src_w, dst_w, vm_w, ex_b, ridx_w, den_sh):
        cid = lax.axis_index("c")
        sid = lax.axis_index("s")
        wid = sid * 2 + cid
        pltpu.sync_copy(es_h, es_v.at[pl.ds(0, nh)])
        pltpu.sync_copy(ed_h, ed_v.at[pl.ds(0, nh)])
        pltpu.sync_copy(zeros_h.at[pl.ds(0, dpr), :], den_v)
        pltpu.sync_copy(src_h.at[wid], src_w)
        pltpu.sync_copy(dst_h.at[wid], dst_w)
        pltpu.sync_copy(vm_h.at[wid], vm_w)
        pltpu.sync_copy(rows_idx_h.at[pl.ds(0, dpr // 128)], ridx_w)

        def grp(g, b):
            off = g * 16
            srcv = src_w[b, pl.ds(off, 16)]
            dstv = dst_w[b, pl.ds(off, 16)]
            vmv = vm_w[b, pl.ds(off, 16)]
            for h in range(heads):
                esv = plsc.load_gather(es_v, [srcv * heads + h])
                edv = plsc.load_gather(ed_v, [dstv * heads + h])
                e = esv + edv
                e = jnp.maximum(e, 0.2 * e)
                ex = jnp.exp(e) * vmv
                ex_b[h, pl.ds(off, 16)] = ex
                idx = dstv * heads + h
                plsc.addupdate_scatter(den_v, [idx >> 7, idx & 127], ex)
            return b

        def batch(b, _):
            lax.fori_loop(0, _EB // 16, grp, b)
            for h in range(heads):
                pltpu.sync_copy(ex_b.at[h], ex_h.at[h, wid, b])
            return 0

        lax.fori_loop(0, _NBS, batch, 0)
        pltpu.sync_copy(zeros_h.at[pl.ds(0, stripe), :],
                        den_sh.at[pl.ds(sid * stripe, stripe), :])
        plsc.subcore_barrier()
        for blk in range(dpr // 128):
            pltpu.sync_copy(den_v.at[pl.ds(blk * _EB, _EB), :],
                            den_sh.at[ridx_w.at[blk]], add=True)
        plsc.subcore_barrier()
        pltpu.sync_copy(den_sh.at[pl.ds(sid * stripe, stripe), :],
                        den_h.at[cid, pl.ds(sid * stripe, stripe), :])

    return sca


@functools.cache
def _make_scal(heads):
    dpr = _dpr(heads)
    mesh = plsc.VectorSubcoreMesh(core_axis_name="c", subcore_axis_name="s")

    @functools.partial(
        pl.kernel, mesh=mesh,
        compiler_params=pltpu.CompilerParams(needs_layout_passes=False),
        out_type=jax.ShapeDtypeStruct((heads, _NW, _NBS, _EB), jnp.float32),
        scratch_types=[
            pltpu.VMEM((dpr, 128), jnp.float32),
            pltpu.VMEM((16, 128), jnp.float32),
            pltpu.VMEM((_NBS, _EB), jnp.int32),
            pltpu.VMEM((_NBS, _EB), jnp.float32),
            pltpu.VMEM((_NBS, _EB), jnp.float32),
        ],
    )
    def scal(ex_h, den_h, dst_h, al_h,
             den_a, den_blk, dst_w, ex_w, al_w):
        cid = lax.axis_index("c")
        sid = lax.axis_index("s")
        wid = sid * 2 + cid
        pltpu.sync_copy(den_h.at[0], den_a)
        pltpu.sync_copy(dst_h.at[wid], dst_w)

        def dsum(blk, _):
            pltpu.sync_copy(den_h.at[1, pl.ds(blk * 16, 16)], den_blk)
            for r in range(16):
                for q in range(8):
                    sl = pl.ds(q * 16, 16)
                    den_a[blk * 16 + r, sl] = (den_a[blk * 16 + r, sl]
                                               + den_blk[r, sl])
            return 0

        lax.fori_loop(0, dpr // 16, dsum, 0)
        for h in range(heads):
            pltpu.sync_copy(ex_h.at[h, wid], ex_w)

            def agrp(g, b):
                off = g * 16
                dstv = dst_w[b, pl.ds(off, 16)]
                exv = ex_w[b, pl.ds(off, 16)]
                idx = dstv * heads + h
                denv = plsc.load_gather(den_a, [idx >> 7, idx & 127])
                al_w[b, pl.ds(off, 16)] = exv / (denv + 1e-16)
                return b

            def abatch(b, _):
                lax.fori_loop(0, _EB // 16, agrp, b)
                return 0

            lax.fori_loop(0, _NBS, abatch, 0)
            pltpu.sync_copy(al_w, al_h.at[h, wid])

    return scal


@functools.cache
def _make_scb(nchunks, heads):
    cs = nchunks // 2
    hdiv = nchunks // heads
    rows_pt = _NP // 16
    mesh = plsc.VectorSubcoreMesh(core_axis_name="c", subcore_axis_name="s")

    @functools.partial(
        pl.kernel, mesh=mesh,
        compiler_params=pltpu.CompilerParams(needs_layout_passes=False),
        out_type=jax.ShapeDtypeStruct((nchunks, _NP, 128), jnp.float32),
        scratch_types=[
            pltpu.VMEM((_NBS, _EB), jnp.int32),
            pltpu.VMEM((_NBS, _EB), jnp.int32),
            pltpu.VMEM((2, _EB), jnp.float32),
            pltpu.VMEM((_EB, 128), jnp.float32),
            pltpu.VMEM((_EB, 128), jnp.float32),
            pltpu.VMEM_SHARED((_NP, 128), jnp.float32),
            pltpu.SemaphoreType.DMA,
            pltpu.SemaphoreType.DMA,
        ],
    )
    def scb(h3_h, src_h, dst_h, al_h, z2_h, o3_h,
            src_w, dst_w, al_b, rows_a, rows_b, acc_sh, sem_a, sem_b):
        cid = lax.axis_index("c")
        sid = lax.axis_index("s")

        def chunk(k, _):
            c = cid * cs + k
            h = c // hdiv
            pltpu.sync_copy(z2_h, acc_sh.at[pl.ds(sid * rows_pt, rows_pt)])
            plsc.subcore_barrier()

            def half(b1, _):
                pltpu.sync_copy(src_h.at[2 * sid + b1], src_w)
                pltpu.sync_copy(dst_h.at[2 * sid + b1], dst_w)
                hc = h3_h.at[c]
                pltpu.async_copy(hc.at[src_w.at[0]], rows_a, sem_a)

                def do(b, rows_v, sem):
                    pltpu.make_async_copy(hc.at[src_w.at[b]], rows_v,
                                          sem).wait()
                    pltpu.sync_copy(al_h.at[h, 2 * sid + b1, pl.ds(b, 1)],
                                    al_b.at[pl.ds(0, 1)])

                    def scale(g, _):
                        av16 = al_b[0, pl.ds(g * 16, 16)]
                        for j in range(16):
                            avv = jnp.full((16,), av16[j], jnp.float32)
                            e = g * 16 + j
                            for q in range(8):
                                rows_v[e, pl.ds(q * 16, 16)] = (
                                    rows_v[e, pl.ds(q * 16, 16)] * avv)
                        return 0

                    lax.fori_loop(0, _EB // 16, scale, 0)
                    pltpu.sync_copy(rows_v, acc_sh.at[dst_w.at[b]],
                                    add=True)

                def pair(i, _):
                    pltpu.async_copy(hc.at[src_w.at[2 * i + 1]], rows_b,
                                     sem_b)
                    do(2 * i, rows_a, sem_a)
                    pltpu.async_copy(hc.at[src_w.at[2 * i + 2]], rows_a,
                                     sem_a)
                    do(2 * i + 1, rows_b, sem_b)
                    return 0

                lax.fori_loop(0, (_NBS - 1) // 2, pair, 0)
                do(_NBS - 1, rows_a, sem_a)
                return 0

            lax.fori_loop(0, 2, half, 0)
            plsc.subcore_barrier()
            pltpu.sync_copy(acc_sh.at[pl.ds(sid * rows_pt, rows_pt)],
                            o3_h.at[c, pl.ds(sid * rows_pt, rows_pt)])
            plsc.subcore_barrier()
            return 0

        lax.fori_loop(0, cs, chunk, 0)

    return scb


@functools.cache
def _make_scc():
    mesh = plsc.VectorSubcoreMesh(core_axis_name="c", subcore_axis_name="s")

    @functools.partial(
        pl.kernel, mesh=mesh,
        compiler_params=pltpu.CompilerParams(needs_layout_passes=False),
        out_type=jax.ShapeDtypeStruct((_FP,), jnp.float32),
        scratch_types=[
            pltpu.VMEM((_NBF, _FB), jnp.int32),
            pltpu.VMEM((_NBF, _FB), jnp.int32),
            pltpu.VMEM((_FB, 256), jnp.float32),
            pltpu.VMEM((_FB, 256), jnp.float32),
            pltpu.VMEM((_FB * 16,), jnp.float32),
            pltpu.VMEM((_FPW,), jnp.float32),
            pltpu.SemaphoreType.DMA,
            pltpu.SemaphoreType.DMA,
        ],
    )
    def scc(h2_h, e0_h, e1_h, sig_h,
            e0_w, e1_w, buf_a, buf_b, pb, out_w, sem_a, sem_b):
        cid = lax.axis_index("c")
        sid = lax.axis_index("s")
        wid = sid * 2 + cid
        pltpu.sync_copy(e0_h.at[wid], e0_w)
        pltpu.sync_copy(e1_h.at[wid], e1_w)

        def batch(b, _):
            da = pltpu.async_copy(h2_h.at[e0_w.at[b]], buf_a, sem_a)
            db = pltpu.async_copy(h2_h.at[e1_w.at[b]], buf_b, sem_b)
            da.wait()
            db.wait()

            def dot(e, _):
                acc = jnp.zeros((16,), jnp.float32)
                for q in range(16):
                    acc = acc + (buf_a[e, pl.ds(q * 16, 16)]
                                 * buf_b[e, pl.ds(q * 16, 16)])
                pb[pl.ds(e * 16, 16)] = acc
                return 0

            lax.fori_loop(0, _FB, dot, 0, unroll=4)

            def red(g, _):
                row0 = (g * 16 + lax.iota(jnp.int32, 16)) * 16
                s = jnp.zeros((16,), jnp.float32)
                for q in range(16):
                    s = s + plsc.load_gather(pb, [row0 + q])
                sig = 1.0 / (1.0 + jnp.exp(-s))
                out_w[pl.ds(b * _FB + g * 16, 16)] = sig
                return 0

            lax.fori_loop(0, _FB // 16, red, 0)
            return 0

        lax.fori_loop(0, _NBF, batch, 0)
        pltpu.sync_copy(out_w, sig_h.at[pl.ds(wid * _FPW, _FPW)])

    return scc


def _pad_idx(x, total):
    out = jnp.zeros((total,), jnp.int32)
    return out.at[:x.shape[0]].set(x.astype(jnp.int32))


def _group_edges(src, dst):
    ge = (dst // 640).astype(jnp.int32)
    cnt = jnp.bincount(ge, length=_NG)
    bnd = jnp.cumsum(_GS - cnt)
    gpad = jnp.searchsorted(bnd, jnp.arange(_NS - _ET), side="right"
                            ).astype(jnp.int32)
    g_all = jnp.concatenate([ge, gpad])
    ispad = jnp.concatenate([jnp.zeros((_ET,), jnp.int32),
                             jnp.ones((_NS - _ET,), jnp.int32)])
    perm = jnp.argsort(g_all * 2 + ispad, stable=True)
    src_all = jnp.concatenate([src.astype(jnp.int32),
                               jnp.zeros((_NS - _ET,), jnp.int32)])
    dst_all = jnp.concatenate([dst.astype(jnp.int32), gpad * 640])
    vm_all = jnp.concatenate([jnp.ones((_ET,), jnp.float32),
                              jnp.zeros((_NS - _ET,), jnp.float32)])
    shape = (_NW, _NBS, _EB)
    return (src_all[perm].reshape(shape), dst_all[perm].reshape(shape),
            vm_all[perm].reshape(shape))


def kernel(Features, A, E, W1, a1_src, a1_dst, b1, W2, a2_src, a2_dst, b2):
    loops = jnp.arange(_N, dtype=A.dtype)
    src = jnp.concatenate([A[0], loops])
    dst = jnp.concatenate([A[1], loops])
    src3, dst3, vm3 = _group_edges(src, dst)
    e03 = _pad_idx(E[0], _FP).reshape(_NW, _NBF, _FB)
    e13 = _pad_idx(E[1], _FP).reshape(_NW, _NBF, _FB)
    zeros_dn = jnp.zeros((256, 128), jnp.float32)
    zeros2 = jnp.zeros((_NP // 16, 128), jnp.float32)
    rows_idx = jnp.arange(256, dtype=jnp.int32).reshape(2, 128)

    h3_1, es1, ed1 = _tc1(Features, W1, a1_src, a1_dst)
    ex1, den1 = _make_sca(3)(es1.reshape(-1), ed1.reshape(-1),
                             src3, dst3, vm3, zeros_dn, rows_idx)
    al1 = _make_scal(3)(ex1, den1, dst3)
    out1 = _make_scb(6, 3)(h3_1, src3, dst3, al1, zeros2)
    h23, es2, ed2 = _tc2(out1, b1, W2, a2_src, a2_dst)
    ex2, den2 = _make_sca(1)(es2.reshape(-1), ed2.reshape(-1),
                             src3, dst3, vm3, zeros_dn, rows_idx)
    al2 = _make_scal(1)(ex2, den2, dst3)
    out2 = _make_scb(2, 1)(h23, src3, dst3, al2, zeros2)
    h2 = _tc3(out2, b2)
    sig = _make_scc()(h2, e03, e13)
    return _tc4(sig)[:_E]
